# baseline (speedup 1.0000x reference)
import functools

import jax
import jax.numpy as jnp
from jax import lax
from jax.experimental import pallas as pl
from jax.experimental.pallas import tpu as pltpu
from jax.experimental.pallas import tpu_sc as plsc

_NUM_CONTEXTS = 32
_NUM_GENS = 96
_MAX_DRAFT = 3
_ROWS = _NUM_CONTEXTS + _NUM_GENS * (_MAX_DRAFT + 1)
_VOCAB = 100000

_TC_ROWS = 224
_SC_ROWS = _ROWS - _TC_ROWS

_VB = 2048
_NB = -(-_VOCAB // _VB)
_TAIL = _VOCAB - (_NB - 1) * _VB


def _tc_argmax_body(x_ref, tt_ref, val_ref, m_scr, a_scr):
    j = pl.program_id(0)

    def _reduce(x):
        col = jax.lax.broadcasted_iota(jnp.int32, (_TC_ROWS, _VB), 1)
        lmax = jnp.max(x, axis=1, keepdims=True)
        larg = jnp.min(jnp.where(x == lmax, col, _VB), axis=1, keepdims=True)
        return lmax, larg + j * _VB

    def _accum(lmax, larg):
        better = lmax > m_scr[...]
        m_scr[...] = jnp.where(better, lmax, m_scr[...])
        a_scr[...] = jnp.where(better, larg, a_scr[...])

    @pl.when(j == 0)
    def _init():
        lmax, larg = _reduce(x_ref[...])
        m_scr[...] = lmax
        a_scr[...] = larg

    @pl.when((j > 0) & (j < _NB - 1))
    def _mid():
        _accum(*_reduce(x_ref[...]))

    @pl.when(j == _NB - 1)
    def _fin():
        col = jax.lax.broadcasted_iota(jnp.int32, (_TC_ROWS, _VB), 1)
        x = jnp.where(col < _TAIL, x_ref[...], -jnp.inf)
        _accum(*_reduce(x))
        tt_ref[...] = a_scr[...]
        val_ref[...] = m_scr[...]


def _tc_argmax(logits):
    return pl.pallas_call(
        _tc_argmax_body,
        grid=(_NB,),
        in_specs=[pl.BlockSpec((_TC_ROWS, _VB), lambda j: (0, j))],
        out_specs=[
            pl.BlockSpec((_TC_ROWS, 1), lambda j: (0, 0)),
            pl.BlockSpec((_TC_ROWS, 1), lambda j: (0, 0)),
        ],
        out_shape=[
            jax.ShapeDtypeStruct((_TC_ROWS, 1), jnp.int32),
            jax.ShapeDtypeStruct((_TC_ROWS, 1), jnp.float32),
        ],
        scratch_shapes=[
            pltpu.VMEM((_TC_ROWS, 1), jnp.float32),
            pltpu.VMEM((_TC_ROWS, 1), jnp.int32),
        ],
    )(logits)


_NW = 32
_RPW = _SC_ROWS // _NW
_CH = 10000
_NCH_ROW = _VOCAB // _CH
_NCH = _RPW * _NCH_ROW
_LANES = 16


def _sc_argmax_body(logits_hbm, tt_out, val_out, buf, mref, aref, iref, vref,
                    sem0, sem1):
    cid = lax.axis_index("c")
    sid = lax.axis_index("s")
    wid = sid * 2 + cid
    base_row = _TC_ROWS + wid * _RPW
    viota = lax.broadcasted_iota(jnp.int32, (_LANES,), 0)

    mref[...] = jnp.full((_LANES,), -jnp.inf, dtype=jnp.float32)
    aref[...] = jnp.zeros((_LANES,), dtype=jnp.int32)

    pltpu.async_copy(logits_hbm.at[base_row, pl.ds(0, _CH)], buf.at[0], sem0)

    def _scan(b, coff):
        vbase = viota + coff

        def inner(i, mc):
            vmax, varg = mc
            v = b[pl.ds(i * _LANES, _LANES)]
            vcur = vbase + i * _LANES
            take = v > vmax
            return jnp.where(take, v, vmax), jnp.where(take, vcur, varg)

        m1, a1 = lax.fori_loop(0, _CH // _LANES, inner,
                               (mref[...], aref[...]), unroll=8)
        mref[...] = m1
        aref[...] = a1

    def chunk_body(c, carry):
        nxt = c + 1
        nrow = base_row + nxt // _NCH_ROW
        noff = (nxt % _NCH_ROW) * _CH

        @pl.when((nxt < _NCH) & (nxt % 2 == 0))
        def _start_even():
            pltpu.async_copy(logits_hbm.at[nrow, pl.ds(noff, _CH)],
                             buf.at[0], sem0)

        @pl.when((nxt < _NCH) & (nxt % 2 == 1))
        def _start_odd():
            pltpu.async_copy(logits_hbm.at[nrow, pl.ds(noff, _CH)],
                             buf.at[1], sem1)

        coff = (c % _NCH_ROW) * _CH

        @pl.when(c % 2 == 0)
        def _scan_even():
            pltpu.make_async_copy(logits_hbm.at[base_row, pl.ds(0, _CH)],
                                  buf.at[0], sem0).wait()
            _scan(buf.at[0], coff)

        @pl.when(c % 2 == 1)
        def _scan_odd():
            pltpu.make_async_copy(logits_hbm.at[base_row, pl.ds(0, _CH)],
                                  buf.at[1], sem1).wait()
            _scan(buf.at[1], coff)

        @pl.when(nxt % _NCH_ROW == 0)
        def _fin_row():
            r = c // _NCH_ROW
            m = mref[...]
            a = aref[...]
            best = jnp.max(m)
            bidx = jnp.min(jnp.where(m == best, a, _VOCAB))
            lane = viota == r
            vref[...] = jnp.where(lane, best, vref[...])
            iref[...] = jnp.where(lane, bidx, iref[...])
            mref[...] = jnp.full((_LANES,), -jnp.inf, dtype=jnp.float32)
            aref[...] = jnp.zeros((_LANES,), dtype=jnp.int32)

        return carry

    lax.fori_loop(0, _NCH, chunk_body, 0)
    pltpu.sync_copy(iref, tt_out.at[wid])
    pltpu.sync_copy(vref, val_out.at[wid])


@functools.partial(
    pl.kernel,
    out_type=[
        jax.ShapeDtypeStruct((_NW, _LANES), jnp.int32),
        jax.ShapeDtypeStruct((_NW, _LANES), jnp.float32),
    ],
    mesh=plsc.VectorSubcoreMesh(core_axis_name="c", subcore_axis_name="s"),
    compiler_params=pltpu.CompilerParams(use_tc_tiling_on_sc=False,
                                         needs_layout_passes=False),
    scratch_types=[
        pltpu.VMEM((2, _CH), jnp.float32),
        pltpu.VMEM((_LANES,), jnp.float32),
        pltpu.VMEM((_LANES,), jnp.int32),
        pltpu.VMEM((_LANES,), jnp.int32),
        pltpu.VMEM((_LANES,), jnp.float32),
        pltpu.SemaphoreType.DMA,
        pltpu.SemaphoreType.DMA,
    ],
)
def _sc_argmax(logits_hbm, tt_out, val_out, buf, mref, aref, iref, vref,
               sem0, sem1):
    _sc_argmax_body(logits_hbm, tt_out, val_out, buf, mref, aref, iref, vref,
                    sem0, sem1)


def _accept_body(gen_t_ref, draft_ref, acc_ref):
    gen_t = gen_t_ref[...]
    draft = draft_ref[...]
    m = (draft == gen_t[:, :_MAX_DRAFT]).astype(jnp.int32)
    run = m[:, 0:1]
    total = run
    for k in range(1, _MAX_DRAFT):
        run = run * m[:, k:k + 1]
        total = total + run
    acc_ref[...] = 1 + total


def _accept(gen_t, draft):
    return pl.pallas_call(
        _accept_body,
        out_shape=jax.ShapeDtypeStruct((_NUM_GENS, 1), jnp.int32),
    )(gen_t, draft)


@jax.jit
def kernel(logits, draft_tokens):
    if logits.ndim == 1:
        logits = logits[None, :]
    draft_tokens = draft_tokens.astype(jnp.int32)

    tt_tc, val_tc = _tc_argmax(logits)
    tt_sc, val_sc = _sc_argmax(logits)

    target_tokens = jnp.concatenate(
        [tt_tc[:, 0], tt_sc[:, :_RPW].reshape(_SC_ROWS)])
    accepted_values = jnp.concatenate(
        [val_tc[:, 0], val_sc[:, :_RPW].reshape(_SC_ROWS)])

    gen_t = target_tokens[_NUM_CONTEXTS:].reshape(_NUM_GENS, _MAX_DRAFT + 1)
    num_acc_gen = _accept(gen_t, draft_tokens)

    ctx_accepted = jnp.concatenate(
        [target_tokens[:_NUM_CONTEXTS, None],
         jnp.zeros((_NUM_CONTEXTS, _MAX_DRAFT), dtype=jnp.int32)], axis=1)
    accepted_tokens = jnp.concatenate([ctx_accepted, gen_t], axis=0)
    num_accepted = jnp.concatenate(
        [jnp.ones((_NUM_CONTEXTS,), dtype=jnp.int32), num_acc_gen[:, 0]], axis=0)
    return accepted_tokens, num_accepted, accepted_values
